# Initial kernel scaffold; baseline (speedup 1.0000x reference)
import jax
import jax.numpy as jnp
from jax.experimental import pallas as pl


def kernel(emb_cat):
    raise NotImplementedError("write your pallas kernel here")


# speedup vs baseline: 17.5747x; 17.5747x over previous
import functools

import jax
import jax.numpy as jnp
from jax.experimental import pallas as pl

N = 1024
K = 102
INT_MIN = -2147483648


def _loss_kernel(emb_ref, out_ref):
    e = emb_ref[...]
    norm = jnp.sqrt(jnp.sum(e * e, axis=1, keepdims=True))
    z = e / jnp.maximum(norm, 1e-12)
    rnorm = jnp.sqrt(jnp.sum(z * z, axis=1, keepdims=True))
    r = z / jnp.maximum(rnorm, 1e-8)

    s = jax.lax.dot_general(
        r, r, (((1,), (1,)), ((), ())), preferred_element_type=jnp.float32
    )

    bits = jax.lax.bitcast_convert_type(s, jnp.int32)
    keys = jnp.where(bits >= 0, bits, INT_MIN - bits)

    lo0 = jnp.full((N, 1), -1073741824, dtype=jnp.int32)
    hi0 = jnp.full((N, 1), 1073741823, dtype=jnp.int32)

    def body(_, carry):
        lo, hi = carry
        mid = lo + ((hi - lo) >> 1)
        cnt = jnp.sum((keys <= mid).astype(jnp.int32), axis=1, keepdims=True)
        take = cnt >= K
        return jnp.where(take, lo, mid + 1), jnp.where(take, mid, hi)

    kth, _ = jax.lax.fori_loop(0, 31, body, (lo0, hi0))

    kth_bits = jnp.where(kth >= 0, kth, INT_MIN - kth)
    v_kth = jax.lax.bitcast_convert_type(kth_bits, jnp.float32)

    below = keys < kth
    cnt_below = jnp.sum(below.astype(jnp.int32), axis=1, keepdims=True)
    expm = jnp.exp(2.0 * s)
    sum_below = jnp.sum(jnp.where(below, expm, 0.0), axis=1, keepdims=True)
    denom = sum_below + (K - cnt_below).astype(jnp.float32) * jnp.exp(2.0 * v_kth)

    rows = jax.lax.broadcasted_iota(jnp.int32, (N, N), 0)
    cols = jax.lax.broadcasted_iota(jnp.int32, (N, N), 1)
    shift = jnp.where(rows < N // 2, rows + N // 2, rows - N // 2)
    pos = jnp.sum(jnp.where(cols == shift, s, 0.0), axis=1, keepdims=True)

    loss_rows = jnp.log(denom) - 2.0 * pos
    loss = jnp.sum(loss_rows) * (1.0 / N)
    out_ref[...] = jnp.full((1, 1), loss, dtype=jnp.float32)


@jax.jit
def kernel(emb_cat):
    out = pl.pallas_call(
        _loss_kernel,
        out_shape=jax.ShapeDtypeStruct((1, 1), jnp.float32),
    )(emb_cat)
    return out[0, 0]


# speedup vs baseline: 32.5369x; 1.8513x over previous
import functools

import jax
import jax.numpy as jnp
from jax.experimental import pallas as pl

N = 1024
K = 102
INT_MIN = -2147483648


def _loss_kernel(emb_ref, out_ref):
    e = emb_ref[...]
    norm = jnp.sqrt(jnp.sum(e * e, axis=1, keepdims=True))
    z = e / jnp.maximum(norm, 1e-12)
    rnorm = jnp.sqrt(jnp.sum(z * z, axis=1, keepdims=True))
    r = z / jnp.maximum(rnorm, 1e-8)

    s = jax.lax.dot_general(
        r, r, (((1,), (1,)), ((), ())), preferred_element_type=jnp.float32
    )

    lo0 = jnp.full((N, 1), -1.5, dtype=jnp.float32)
    hi0 = jnp.full((N, 1), 1.5, dtype=jnp.float32)

    def body(_, carry):
        lo, hi = carry
        mid = 0.5 * (lo + hi)
        cnt = jnp.sum((s <= mid).astype(jnp.float32), axis=1, keepdims=True)
        take = cnt >= K
        return jnp.where(take, lo, mid), jnp.where(take, mid, hi)

    lo, hi = jax.lax.fori_loop(0, 16, body, (lo0, hi0))

    below = s <= lo
    cnt_below = jnp.sum(below.astype(jnp.float32), axis=1, keepdims=True)
    expm = jnp.exp(2.0 * s)
    sum_below = jnp.sum(jnp.where(below, expm, 0.0), axis=1, keepdims=True)
    denom = sum_below + (K - cnt_below) * jnp.exp(lo + hi)

    rows = jax.lax.broadcasted_iota(jnp.int32, (N, N), 0)
    cols = jax.lax.broadcasted_iota(jnp.int32, (N, N), 1)
    shift = jnp.where(rows < N // 2, rows + N // 2, rows - N // 2)
    pos = jnp.sum(jnp.where(cols == shift, s, 0.0), axis=1, keepdims=True)

    loss_rows = jnp.log(denom) - 2.0 * pos
    loss = jnp.sum(loss_rows) * (1.0 / N)
    out_ref[...] = jnp.full((1, 1), loss, dtype=jnp.float32)


@jax.jit
def kernel(emb_cat):
    out = pl.pallas_call(
        _loss_kernel,
        out_shape=jax.ShapeDtypeStruct((1, 1), jnp.float32),
    )(emb_cat)
    return out[0, 0]


# speedup vs baseline: 33.1634x; 1.0193x over previous
import functools

import jax
import jax.numpy as jnp
from jax.experimental import pallas as pl

N = 1024
K = 102
INT_MIN = -2147483648


def _loss_kernel(emb_ref, out_ref):
    e = emb_ref[...]
    norm = jnp.sqrt(jnp.sum(e * e, axis=1, keepdims=True))
    z = e / jnp.maximum(norm, 1e-12)
    rnorm = jnp.sqrt(jnp.sum(z * z, axis=1, keepdims=True))
    r = z / jnp.maximum(rnorm, 1e-8)

    rb = r.astype(jnp.bfloat16)
    s = jax.lax.dot_general(
        rb, rb, (((1,), (1,)), ((), ())), preferred_element_type=jnp.float32
    )

    lo0 = jnp.full((N, 1), -1.5, dtype=jnp.float32)
    hi0 = jnp.full((N, 1), 1.5, dtype=jnp.float32)

    def body(_, carry):
        lo, hi = carry
        mid = 0.5 * (lo + hi)
        cnt = jnp.sum((s <= mid).astype(jnp.float32), axis=1, keepdims=True)
        take = cnt >= K
        return jnp.where(take, lo, mid), jnp.where(take, mid, hi)

    lo, hi = jax.lax.fori_loop(0, 16, body, (lo0, hi0))

    below = s <= lo
    cnt_below = jnp.sum(below.astype(jnp.float32), axis=1, keepdims=True)
    expm = jnp.exp(2.0 * s)
    sum_below = jnp.sum(jnp.where(below, expm, 0.0), axis=1, keepdims=True)
    denom = sum_below + (K - cnt_below) * jnp.exp(lo + hi)

    rows = jax.lax.broadcasted_iota(jnp.int32, (N, N), 0)
    cols = jax.lax.broadcasted_iota(jnp.int32, (N, N), 1)
    shift = jnp.where(rows < N // 2, rows + N // 2, rows - N // 2)
    pos = jnp.sum(jnp.where(cols == shift, s, 0.0), axis=1, keepdims=True)

    loss_rows = jnp.log(denom) - 2.0 * pos
    loss = jnp.sum(loss_rows) * (1.0 / N)
    out_ref[...] = jnp.full((1, 1), loss, dtype=jnp.float32)


@jax.jit
def kernel(emb_cat):
    out = pl.pallas_call(
        _loss_kernel,
        out_shape=jax.ShapeDtypeStruct((1, 1), jnp.float32),
    )(emb_cat)
    return out[0, 0]


# speedup vs baseline: 39.3699x; 1.1871x over previous
import functools

import jax
import jax.numpy as jnp
from jax.experimental import pallas as pl

N = 1024
K = 102
T_ITERS = 12


def _loss_kernel(emb_ref, out_ref):
    e = emb_ref[...]
    eb = e.astype(jnp.bfloat16)
    g = jax.lax.dot_general(
        eb, eb, (((1,), (1,)), ((), ())), preferred_element_type=jnp.float32
    )

    rows = jax.lax.broadcasted_iota(jnp.int32, (N, N), 0)
    cols = jax.lax.broadcasted_iota(jnp.int32, (N, N), 1)
    diag = jnp.sum(jnp.where(rows == cols, g, 0.0), axis=1, keepdims=True)
    nrm = jnp.sqrt(diag)
    inv = 1.0 / jnp.maximum(nrm, 1e-12)

    u = g * inv.reshape(1, N)

    lo0 = -1.5 * nrm
    hi0 = 1.5 * nrm

    def body(_, carry):
        lo, hi = carry
        mid = 0.5 * (lo + hi)
        cnt = jnp.sum((u <= mid).astype(jnp.float32), axis=1, keepdims=True)
        take = cnt >= K
        return jnp.where(take, lo, mid), jnp.where(take, mid, hi)

    lo, hi = jax.lax.fori_loop(0, T_ITERS, body, (lo0, hi0))

    below = u <= lo
    cnt_below = jnp.sum(below.astype(jnp.float32), axis=1, keepdims=True)
    expm = jnp.exp((2.0 * inv) * u)
    sum_below = jnp.sum(jnp.where(below, expm, 0.0), axis=1, keepdims=True)
    denom = sum_below + (K - cnt_below) * jnp.exp(inv * (lo + hi))

    shift = jnp.where(rows < N // 2, rows + N // 2, rows - N // 2)
    pos = inv * jnp.sum(jnp.where(cols == shift, u, 0.0), axis=1, keepdims=True)

    loss_rows = jnp.log(denom) - 2.0 * pos
    loss = jnp.sum(loss_rows) * (1.0 / N)
    out_ref[...] = jnp.full((1, 1), loss, dtype=jnp.float32)


@jax.jit
def kernel(emb_cat):
    out = pl.pallas_call(
        _loss_kernel,
        out_shape=jax.ShapeDtypeStruct((1, 1), jnp.float32),
    )(emb_cat)
    return out[0, 0]
